# baseline (speedup 1.0000x reference)
import functools
import jax
import jax.numpy as jnp
from jax import lax
from jax.experimental import pallas as pl
from jax.experimental.pallas import tpu as pltpu
from jax.experimental.pallas import tpu_sc as plsc

NC, NS, L = 2, 16, 16
NW = NC * NS
BATCH = 4
SEQ = 8192
D = 1024
SPW = SEQ // NW
CH = 4
NCH = SPW // CH
NP = 4
NSLOT = 2
UNROLL = 8


def _sc_add(w2d, pe2d):
    mesh = plsc.VectorSubcoreMesh(core_axis_name="c", subcore_axis_name="s",
                                  num_cores=NC, num_subcores=NS)

    @functools.partial(
        pl.kernel,
        out_type=jax.ShapeDtypeStruct((BATCH * SEQ, D), jnp.float32),
        mesh=mesh,
        scratch_types=(
            [pltpu.VMEM((BATCH * CH, D), jnp.float32) for _ in range(NP)]
            + [pltpu.VMEM((CH, D), jnp.float32) for _ in range(NP)]
            + [pltpu.VMEM_SHARED((NS, NSLOT, BATCH * CH, D), jnp.float32)]
            + [pltpu.SemaphoreType.DMA for _ in range(2 * NP + 2 * NSLOT)]
        ),
    )
    def k(w_hbm, pe_hbm, out_hbm, *scratch):
        wbufs = list(scratch[:NP])
        pbufs = list(scratch[NP:2 * NP])
        smem_out = scratch[2 * NP]
        sems = list(scratch[2 * NP + 1:])
        swl = sems[:NP]
        spl = sems[NP:2 * NP]
        sts = sems[2 * NP:2 * NP + NSLOT]
        ssh = sems[2 * NP + NSLOT:]

        wid = lax.axis_index("s") * NC + lax.axis_index("c")
        sid = lax.axis_index("s")
        s0 = wid * SPW

        def issue_loads(g, h):
            r0 = s0 + g * CH
            pltpu.async_copy(pe_hbm.at[pl.ds(r0, CH)], pbufs[h], spl[h])
            for b in range(BATCH):
                pltpu.async_copy(
                    w_hbm.at[pl.ds(b * SEQ + r0, CH)],
                    wbufs[h].at[pl.ds(b * CH, CH)], swl[h])

        def wait_loads(g, h):
            r0 = s0 + g * CH
            pltpu.make_async_copy(
                pe_hbm.at[pl.ds(r0, CH)], pbufs[h], spl[h]).wait()
            for b in range(BATCH):
                pltpu.make_async_copy(
                    w_hbm.at[pl.ds(b * SEQ + r0, CH)],
                    wbufs[h].at[pl.ds(b * CH, CH)], swl[h]).wait()

        def issue_ts(h, sl):
            pltpu.async_copy(wbufs[h], smem_out.at[sid, sl], sts[sl])

        def wait_ts(h, sl):
            pltpu.make_async_copy(
                wbufs[h], smem_out.at[sid, sl], sts[sl]).wait()

        def issue_sh(g, sl):
            r0 = s0 + g * CH
            for b in range(BATCH):
                pltpu.async_copy(
                    smem_out.at[sid, sl, pl.ds(b * CH, CH)],
                    out_hbm.at[pl.ds(b * SEQ + r0, CH)], ssh[sl])

        def wait_sh(g, sl):
            r0 = s0 + g * CH
            for b in range(BATCH):
                pltpu.make_async_copy(
                    smem_out.at[sid, sl, pl.ds(b * CH, CH)],
                    out_hbm.at[pl.ds(b * SEQ + r0, CH)], ssh[sl]).wait()

        def alu(h):
            wb, pb = wbufs[h], pbufs[h]
            for r in range(CH):
                @plsc.parallel_loop(0, D, step=L, unroll=UNROLL)
                def _(i):
                    pv = pb[r, pl.ds(i, L)]
                    for b in range(BATCH):
                        wb[b * CH + r, pl.ds(i, L)] = (
                            wb[b * CH + r, pl.ds(i, L)] + pv)

        def chunk_body(g, h, do_loads=True, do_sh_wait=True):
            sl = h % NSLOT
            sl1 = (sl + 1) % NSLOT
            h1 = (h - 1) % NP
            wait_ts(h1, sl1)
            issue_sh(g - 1, sl1)
            if do_sh_wait:
                wait_sh(g - NSLOT, sl)
            if do_loads:
                issue_loads(g + 2, (h + 2) % NP)
            wait_loads(g, h)
            alu(h)
            issue_ts(h, sl)

        issue_loads(0, 0)
        issue_loads(1, 1)
        issue_loads(2, 2)
        wait_loads(0, 0)
        alu(0)
        issue_ts(0, 0)
        issue_loads(3, 3)
        wait_ts(0, 0)
        issue_sh(0, 0)
        wait_loads(1, 1)
        alu(1)
        issue_ts(1, 1)

        chunk_body(2, 2)
        chunk_body(3, 3)

        def body(j, _):
            for kk in range(4):
                chunk_body(4 * j + 4 + kk, kk)
            return 0

        lax.fori_loop(0, (NCH - 8) // 4, body, 0)

        chunk_body(NCH - 4, (NCH - 4) % NP)
        chunk_body(NCH - 3, (NCH - 3) % NP)
        chunk_body(NCH - 2, (NCH - 2) % NP, do_loads=False)
        chunk_body(NCH - 1, (NCH - 1) % NP, do_loads=False)
        last = NCH - 1
        wait_ts(last % NP, last % NSLOT)
        issue_sh(last, last % NSLOT)
        wait_sh(NCH - 2, (NCH - 2) % NSLOT)
        wait_sh(NCH - 1, (NCH - 1) % NSLOT)

    return k(w2d, pe2d)


def kernel(weights, pe):
    b, s, d = weights.shape
    out = _sc_add(weights.reshape(b * s, d), pe)
    return out.reshape(b, s, d)
